# baseline (speedup 1.0000x reference)
import jax
import jax.numpy as jnp
from jax import lax
from jax.experimental import pallas as pl
from jax.experimental.pallas import tpu as pltpu
from jax.experimental.pallas import tpu_sc as plsc

N_NODES = 10000
N_PART = 1000
N_EDGES = 320000
D_NODE = 128
D_PART = 128
D_ATT = 20

_NW = 32
_EPW = N_EDGES // _NW
_CH = 128
_NFULL = _EPW // _CH
_TAIL = _EPW - _NFULL * _CH
_NPASS = 8
_TW = 1280
_CT = _TW * N_PART
_SLICE = _CT // 10
_ZCH = 5000
_PB = 1000
_NPB = N_NODES // _PB
_BATCH = 6
_NGRP = _NFULL // _BATCH


def _pre_kernel(nodes_ref, ph_ref, pg_ref, kW_ref, kb_ref, qW_ref, qb_ref,
                vW_ref, vb_ref, S_ref, v_ref, q_scr):
    i = pl.program_id(0)

    @pl.when(i == 0)
    def _():
        qW = qW_ref[...]
        q_scr[...] = (ph_ref[...] @ qW[:, :D_PART].T
                      + pg_ref[...] @ qW[:, D_PART:].T + qb_ref[0])

    nodes = nodes_ref[...]
    k = nodes @ kW_ref[...].T + kb_ref[0]
    S_ref[...] = (k @ q_scr[...].T) * (1.0 / jnp.sqrt(jnp.float32(D_ATT)))
    v_ref[...] = nodes @ vW_ref[...].T + vb_ref[0]


def _count_kernel(esrc, edst, zeros, cout, src_c, dst_c, idxbuf, valbuf, zv,
                  ctile, sem):
    c = lax.axis_index("c")
    s = lax.axis_index("s")
    wid = s * 2 + c
    ebase = wid * _EPW
    iota = lax.iota(jnp.int32, 16)

    pltpu.sync_copy(esrc.at[pl.ds(ebase, _EPW)], src_c)
    pltpu.sync_copy(edst.at[pl.ds(ebase, _EPW)], dst_c)
    pltpu.sync_copy(zeros, zv)

    for p in range(_NPASS):
        off = p * _TW

        @pl.when(s < 10)
        def _():
            base = s * _SLICE
            pltpu.sync_copy(zv, ctile.at[pl.ds(base, _ZCH)])
            pltpu.sync_copy(ctile.at[pl.ds(base, 5000)],
                            ctile.at[pl.ds(base + 5000, 5000)])
            pltpu.sync_copy(ctile.at[pl.ds(base, 10000)],
                            ctile.at[pl.ds(base + 10000, 10000)])
            pltpu.sync_copy(ctile.at[pl.ds(base, 20000)],
                            ctile.at[pl.ds(base + 20000, 20000)])
            pltpu.sync_copy(ctile.at[pl.ds(base, 40000)],
                            ctile.at[pl.ds(base + 40000, 40000)])
            pltpu.sync_copy(ctile.at[pl.ds(base, 48000)],
                            ctile.at[pl.ds(base + 80000, 48000)])

        plsc.subcore_barrier()

        def fill(j, coff, n, off=off):
            for t in range(_CH // 16):
                spread = s * 128 + t * 16 + iota
                if t * 16 < n:
                    sl = pl.ds(coff + t * 16, 16)
                    rel = src_c[sl] - off
                    flat = rel * N_PART + dst_c[sl]
                    inb = jnp.logical_and(rel >= 0, rel < _TW)
                    idxbuf[j, pl.ds(t * 16, 16)] = jnp.where(inb, flat, spread)
                    valbuf[j, pl.ds(t * 16, 16)] = jnp.where(inb, 1.0, 0.0)
                else:
                    idxbuf[j, pl.ds(t * 16, 16)] = spread
                    valbuf[j, pl.ds(t * 16, 16)] = jnp.zeros((16,), jnp.float32)

        def group(g, _):
            cps = []
            for b in range(_BATCH):
                j = g * _BATCH + b
                fill(b, j * _CH, _CH)
                cps.append(pltpu.async_copy(
                    valbuf.at[b], ctile.at[idxbuf.at[b]], sem, add=True))
            for cp in cps:
                cp.wait()
            return 0

        lax.fori_loop(0, _NGRP, group, 0)
        fill(0, _NFULL * _CH, _TAIL)
        pltpu.sync_copy(valbuf.at[0], ctile.at[idxbuf.at[0]], add=True)
        plsc.subcore_barrier()

        @pl.when(s < 10)
        def _():
            pltpu.sync_copy(ctile.at[pl.ds(s * _SLICE, _SLICE)],
                            cout.at[c, p, pl.ds(s * _SLICE, _SLICE)])


def _post_kernel(C_ref, S_ref, v_ref, ph_ref, wih_ref, whh_ref, bih_ref,
                 bhh_ref, g_ref, be_ref, w1_ref, b1_ref, w2_ref, b2_ref,
                 out_ref, acc):
    i = pl.program_id(0)

    @pl.when(i == 0)
    def _():
        acc[...] = jnp.zeros_like(acc)

    Cb = C_ref[0] + C_ref[1]
    B = Cb * S_ref[...]
    acc[...] += lax.dot_general(B, v_ref[...], (((0,), (0,)), ((), ())),
                                preferred_element_type=jnp.float32)

    @pl.when(i == _NPB - 1)
    def _():
        ph = ph_ref[...]
        ws = acc[...]
        gi = ws @ wih_ref[...].T + bih_ref[0]
        gh = ph @ whh_ref[...].T + bhh_ref[0]
        i_r, i_z, i_n = (gi[:, :D_PART], gi[:, D_PART:2 * D_PART],
                         gi[:, 2 * D_PART:])
        h_r, h_z, h_n = (gh[:, :D_PART], gh[:, D_PART:2 * D_PART],
                         gh[:, 2 * D_PART:])
        r = jax.nn.sigmoid(i_r + h_r)
        z = jax.nn.sigmoid(i_z + h_z)
        n = jnp.tanh(i_n + r * h_n)
        h = (1.0 - z) * n + z * ph
        mu = jnp.mean(h, axis=-1, keepdims=True)
        var = jnp.mean((h - mu) ** 2, axis=-1, keepdims=True)
        ln = (h - mu) / jnp.sqrt(var + 1e-5) * g_ref[0] + be_ref[0]
        mlp = (jax.nn.relu(ln @ w1_ref[...].T + b1_ref[0]) @ w2_ref[...].T
               + b2_ref[0])
        out_ref[...] = ph + mlp


def kernel(nodes_hidden, particles_hidden, particles_global, edge_src, edge_dst,
           key_W, key_b, query_W, query_b, values_W, values_b,
           gru_W_ih, gru_W_hh, gru_b_ih, gru_b_hh,
           ln_gamma, ln_beta, mlp_W1, mlp_b1, mlp_W2, mlp_b2):
    f32 = jnp.float32
    nb = 10

    S, v = pl.pallas_call(
        _pre_kernel,
        grid=(nb,),
        in_specs=[
            pl.BlockSpec((N_NODES // nb, D_NODE), lambda i: (i, 0)),
            pl.BlockSpec((N_PART, D_PART), lambda i: (0, 0)),
            pl.BlockSpec((N_PART, D_NODE), lambda i: (0, 0)),
            pl.BlockSpec((D_ATT, D_NODE), lambda i: (0, 0)),
            pl.BlockSpec((1, D_ATT), lambda i: (0, 0)),
            pl.BlockSpec((D_ATT, D_PART + D_NODE), lambda i: (0, 0)),
            pl.BlockSpec((1, D_ATT), lambda i: (0, 0)),
            pl.BlockSpec((D_PART, D_NODE), lambda i: (0, 0)),
            pl.BlockSpec((1, D_PART), lambda i: (0, 0)),
        ],
        out_specs=[
            pl.BlockSpec((N_NODES // nb, N_PART), lambda i: (i, 0)),
            pl.BlockSpec((N_NODES // nb, D_PART), lambda i: (i, 0)),
        ],
        out_shape=[
            jax.ShapeDtypeStruct((N_NODES, N_PART), f32),
            jax.ShapeDtypeStruct((N_NODES, D_PART), f32),
        ],
        scratch_shapes=[pltpu.VMEM((N_PART, D_ATT), f32)],
    )(nodes_hidden, particles_hidden, particles_global, key_W,
      key_b.reshape(1, -1), query_W, query_b.reshape(1, -1), values_W,
      values_b.reshape(1, -1))

    count_call = pl.kernel(
        _count_kernel,
        out_type=jax.ShapeDtypeStruct((2, _NPASS, _CT), f32),
        mesh=plsc.VectorSubcoreMesh(core_axis_name="c", subcore_axis_name="s"),
        compiler_params=pltpu.CompilerParams(use_tc_tiling_on_sc=False),
        scratch_types=[
            pltpu.VMEM((_EPW,), jnp.int32),
            pltpu.VMEM((_EPW,), jnp.int32),
            pltpu.VMEM((_BATCH, _CH), jnp.int32),
            pltpu.VMEM((_BATCH, _CH), f32),
            pltpu.VMEM((_ZCH,), f32),
            pltpu.VMEM_SHARED((_CT,), f32),
            pltpu.SemaphoreType.DMA,
        ],
    )
    zeros = jnp.zeros((_ZCH,), f32)
    counts = count_call(edge_src, edge_dst, zeros)
    counts = counts.reshape(2, _NPASS * _TW, N_PART)

    new_hidden = pl.pallas_call(
        _post_kernel,
        grid=(_NPB,),
        in_specs=[
            pl.BlockSpec((2, _PB, N_PART), lambda i: (0, i, 0)),
            pl.BlockSpec((_PB, N_PART), lambda i: (i, 0)),
            pl.BlockSpec((_PB, D_PART), lambda i: (i, 0)),
            pl.BlockSpec((N_PART, D_PART), lambda i: (0, 0)),
            pl.BlockSpec((3 * D_PART, D_PART), lambda i: (0, 0)),
            pl.BlockSpec((3 * D_PART, D_PART), lambda i: (0, 0)),
            pl.BlockSpec((1, 3 * D_PART), lambda i: (0, 0)),
            pl.BlockSpec((1, 3 * D_PART), lambda i: (0, 0)),
            pl.BlockSpec((1, D_PART), lambda i: (0, 0)),
            pl.BlockSpec((1, D_PART), lambda i: (0, 0)),
            pl.BlockSpec((64, D_PART), lambda i: (0, 0)),
            pl.BlockSpec((1, 64), lambda i: (0, 0)),
            pl.BlockSpec((D_PART, 64), lambda i: (0, 0)),
            pl.BlockSpec((1, D_PART), lambda i: (0, 0)),
        ],
        out_specs=pl.BlockSpec((N_PART, D_PART), lambda i: (0, 0)),
        out_shape=jax.ShapeDtypeStruct((N_PART, D_PART), f32),
        scratch_shapes=[pltpu.VMEM((N_PART, D_PART), f32)],
    )(counts, S, v, particles_hidden, gru_W_ih, gru_W_hh,
      gru_b_ih.reshape(1, -1), gru_b_hh.reshape(1, -1),
      ln_gamma.reshape(1, -1), ln_beta.reshape(1, -1),
      mlp_W1, mlp_b1.reshape(1, -1), mlp_W2, mlp_b2.reshape(1, -1))
    return new_hidden


# speedup vs baseline: 8.1608x; 8.1608x over previous
import jax
import jax.numpy as jnp
from jax import lax
from jax.experimental import pallas as pl
from jax.experimental.pallas import tpu as pltpu
from jax.experimental.pallas import tpu_sc as plsc

N_NODES = 10000
N_PART = 1000
N_EDGES = 320000
D_NODE = 128
D_PART = 128
D_ATT = 20

_NW = 32
_EPW = N_EDGES // _NW
_CH = 128
_NFULL = _EPW // _CH
_TAIL = _EPW - _NFULL * _CH
_NPASS = 8
_TW = 1280
_CT = _TW * N_PART
_SLICE = _CT // 10
_ZCH = 8000
_PB = 1000
_NPB = N_NODES // _PB
_BATCH = 6
_NGRP = _NFULL // _BATCH


def _pre_kernel(nodes_ref, ph_ref, pg_ref, kW_ref, kb_ref, qW_ref, qb_ref,
                vW_ref, vb_ref, S_ref, v_ref, q_scr):
    i = pl.program_id(0)

    @pl.when(i == 0)
    def _():
        qW = qW_ref[...]
        q_scr[...] = (ph_ref[...] @ qW[:, :D_PART].T
                      + pg_ref[...] @ qW[:, D_PART:].T + qb_ref[0])

    nodes = nodes_ref[...]
    k = nodes @ kW_ref[...].T + kb_ref[0]
    S_ref[...] = (k @ q_scr[...].T) * (1.0 / jnp.sqrt(jnp.float32(D_ATT)))
    v_ref[...] = nodes @ vW_ref[...].T + vb_ref[0]


def _count_kernel(esrc, edst, zeros, cout, src_c, dst_c, idxbuf, valbuf, zv,
                  ctile, sem):
    c = lax.axis_index("c")
    s = lax.axis_index("s")
    wid = s * 2 + c
    ebase = wid * _EPW
    iota = lax.iota(jnp.int32, 16)

    pltpu.sync_copy(esrc.at[pl.ds(ebase, _EPW)], src_c)
    pltpu.sync_copy(edst.at[pl.ds(ebase, _EPW)], dst_c)
    pltpu.sync_copy(zeros, zv)

    for p in range(_NPASS):
        off = p * _TW

        @pl.when(s < 10)
        def _():
            for r in range(_SLICE // _ZCH):
                pltpu.sync_copy(zv,
                                ctile.at[pl.ds(s * _SLICE + r * _ZCH, _ZCH)])

        plsc.subcore_barrier()

        def fill(j, coff, n, off=off):
            for t in range(_CH // 16):
                spread = s * 128 + t * 16 + iota
                if t * 16 < n:
                    sl = pl.ds(coff + t * 16, 16)
                    rel = src_c[sl] - off
                    flat = rel * N_PART + dst_c[sl]
                    inb = jnp.logical_and(rel >= 0, rel < _TW)
                    idxbuf[j, pl.ds(t * 16, 16)] = jnp.where(inb, flat, spread)
                    valbuf[j, pl.ds(t * 16, 16)] = jnp.where(inb, 1.0, 0.0)
                else:
                    idxbuf[j, pl.ds(t * 16, 16)] = spread
                    valbuf[j, pl.ds(t * 16, 16)] = jnp.zeros((16,), jnp.float32)

        def group(g, _):
            cps = []
            for b in range(_BATCH):
                j = g * _BATCH + b
                fill(b, j * _CH, _CH)
                cps.append(pltpu.async_copy(
                    valbuf.at[b], ctile.at[idxbuf.at[b]], sem, add=True))
            for cp in cps:
                cp.wait()
            return 0

        lax.fori_loop(0, _NGRP, group, 0)
        fill(0, _NFULL * _CH, _TAIL)
        pltpu.sync_copy(valbuf.at[0], ctile.at[idxbuf.at[0]], add=True)
        plsc.subcore_barrier()

        @pl.when(s < 10)
        def _():
            pltpu.sync_copy(ctile.at[pl.ds(s * _SLICE, _SLICE)],
                            cout.at[c, p, pl.ds(s * _SLICE, _SLICE)])


def _post_kernel(C_ref, S_ref, v_ref, ph_ref, wih_ref, whh_ref, bih_ref,
                 bhh_ref, g_ref, be_ref, w1_ref, b1_ref, w2_ref, b2_ref,
                 out_ref, acc):
    i = pl.program_id(0)

    @pl.when(i == 0)
    def _():
        acc[...] = jnp.zeros_like(acc)

    Cb = C_ref[0] + C_ref[1]
    B = Cb * S_ref[...]
    acc[...] += lax.dot_general(B, v_ref[...], (((0,), (0,)), ((), ())),
                                preferred_element_type=jnp.float32)

    @pl.when(i == _NPB - 1)
    def _():
        ph = ph_ref[...]
        ws = acc[...]
        gi = ws @ wih_ref[...].T + bih_ref[0]
        gh = ph @ whh_ref[...].T + bhh_ref[0]
        i_r, i_z, i_n = (gi[:, :D_PART], gi[:, D_PART:2 * D_PART],
                         gi[:, 2 * D_PART:])
        h_r, h_z, h_n = (gh[:, :D_PART], gh[:, D_PART:2 * D_PART],
                         gh[:, 2 * D_PART:])
        r = jax.nn.sigmoid(i_r + h_r)
        z = jax.nn.sigmoid(i_z + h_z)
        n = jnp.tanh(i_n + r * h_n)
        h = (1.0 - z) * n + z * ph
        mu = jnp.mean(h, axis=-1, keepdims=True)
        var = jnp.mean((h - mu) ** 2, axis=-1, keepdims=True)
        ln = (h - mu) / jnp.sqrt(var + 1e-5) * g_ref[0] + be_ref[0]
        mlp = (jax.nn.relu(ln @ w1_ref[...].T + b1_ref[0]) @ w2_ref[...].T
               + b2_ref[0])
        out_ref[...] = ph + mlp


def kernel(nodes_hidden, particles_hidden, particles_global, edge_src, edge_dst,
           key_W, key_b, query_W, query_b, values_W, values_b,
           gru_W_ih, gru_W_hh, gru_b_ih, gru_b_hh,
           ln_gamma, ln_beta, mlp_W1, mlp_b1, mlp_W2, mlp_b2):
    f32 = jnp.float32
    nb = 10

    S, v = pl.pallas_call(
        _pre_kernel,
        grid=(nb,),
        in_specs=[
            pl.BlockSpec((N_NODES // nb, D_NODE), lambda i: (i, 0)),
            pl.BlockSpec((N_PART, D_PART), lambda i: (0, 0)),
            pl.BlockSpec((N_PART, D_NODE), lambda i: (0, 0)),
            pl.BlockSpec((D_ATT, D_NODE), lambda i: (0, 0)),
            pl.BlockSpec((1, D_ATT), lambda i: (0, 0)),
            pl.BlockSpec((D_ATT, D_PART + D_NODE), lambda i: (0, 0)),
            pl.BlockSpec((1, D_ATT), lambda i: (0, 0)),
            pl.BlockSpec((D_PART, D_NODE), lambda i: (0, 0)),
            pl.BlockSpec((1, D_PART), lambda i: (0, 0)),
        ],
        out_specs=[
            pl.BlockSpec((N_NODES // nb, N_PART), lambda i: (i, 0)),
            pl.BlockSpec((N_NODES // nb, D_PART), lambda i: (i, 0)),
        ],
        out_shape=[
            jax.ShapeDtypeStruct((N_NODES, N_PART), f32),
            jax.ShapeDtypeStruct((N_NODES, D_PART), f32),
        ],
        scratch_shapes=[pltpu.VMEM((N_PART, D_ATT), f32)],
    )(nodes_hidden, particles_hidden, particles_global, key_W,
      key_b.reshape(1, -1), query_W, query_b.reshape(1, -1), values_W,
      values_b.reshape(1, -1))

    count_call = pl.kernel(
        _count_kernel,
        out_type=jax.ShapeDtypeStruct((2, _NPASS, _CT), f32),
        mesh=plsc.VectorSubcoreMesh(core_axis_name="c", subcore_axis_name="s"),
        compiler_params=pltpu.CompilerParams(use_tc_tiling_on_sc=False),
        scratch_types=[
            pltpu.VMEM((_EPW,), jnp.int32),
            pltpu.VMEM((_EPW,), jnp.int32),
            pltpu.VMEM((_BATCH, _CH), jnp.int32),
            pltpu.VMEM((_BATCH, _CH), f32),
            pltpu.VMEM((_ZCH,), f32),
            pltpu.VMEM_SHARED((_CT,), f32),
            pltpu.SemaphoreType.DMA,
        ],
    )
    zeros = jnp.zeros((_ZCH,), f32)
    counts = count_call(edge_src, edge_dst, zeros)
    counts = counts.reshape(2, _NPASS * _TW, N_PART)

    new_hidden = pl.pallas_call(
        _post_kernel,
        grid=(_NPB,),
        in_specs=[
            pl.BlockSpec((2, _PB, N_PART), lambda i: (0, i, 0)),
            pl.BlockSpec((_PB, N_PART), lambda i: (i, 0)),
            pl.BlockSpec((_PB, D_PART), lambda i: (i, 0)),
            pl.BlockSpec((N_PART, D_PART), lambda i: (0, 0)),
            pl.BlockSpec((3 * D_PART, D_PART), lambda i: (0, 0)),
            pl.BlockSpec((3 * D_PART, D_PART), lambda i: (0, 0)),
            pl.BlockSpec((1, 3 * D_PART), lambda i: (0, 0)),
            pl.BlockSpec((1, 3 * D_PART), lambda i: (0, 0)),
            pl.BlockSpec((1, D_PART), lambda i: (0, 0)),
            pl.BlockSpec((1, D_PART), lambda i: (0, 0)),
            pl.BlockSpec((64, D_PART), lambda i: (0, 0)),
            pl.BlockSpec((1, 64), lambda i: (0, 0)),
            pl.BlockSpec((D_PART, 64), lambda i: (0, 0)),
            pl.BlockSpec((1, D_PART), lambda i: (0, 0)),
        ],
        out_specs=pl.BlockSpec((N_PART, D_PART), lambda i: (0, 0)),
        out_shape=jax.ShapeDtypeStruct((N_PART, D_PART), f32),
        scratch_shapes=[pltpu.VMEM((N_PART, D_PART), f32)],
    )(counts, S, v, particles_hidden, gru_W_ih, gru_W_hh,
      gru_b_ih.reshape(1, -1), gru_b_hh.reshape(1, -1),
      ln_gamma.reshape(1, -1), ln_beta.reshape(1, -1),
      mlp_W1, mlp_b1.reshape(1, -1), mlp_W2, mlp_b2.reshape(1, -1))
    return new_hidden
